# baseline (speedup 1.0000x reference)
import functools

import jax
import jax.numpy as jnp
from jax import lax
from jax.experimental import pallas as pl
from jax.experimental.pallas import tpu as pltpu
from jax.experimental.pallas import tpu_sc as plsc

_BS = 4
_N = 10000
_SEQ = 9
_INC = 128
_OUTC = 32
_LANES = _BS * _OUTC
_NDOWN = 5000
_NNZ = 20000

_NC = 2
_NS = 16
_NW = _NC * _NS

_CH = 40
_CPW = 8
_NP = _NW * _CPW * _CH
_NCHUNK = _NP // _CH
_CPW0 = 8
_CPW1 = 2 * _CPW - _CPW0

_KCH = 128
_KPT = 5
_NKCH = _NC * _NS * _KPT
_NKP = _NKCH * _KCH
_NDP = 5120


def _mm_body(x_ref, v_ref, o_ref):
    for s in range(_SEQ):
        for bb in range(_BS):
            o_ref[s, :, bb * _OUTC:(bb + 1) * _OUTC] = jnp.dot(
                x_ref[bb], v_ref[s], preferred_element_type=jnp.float32
            )


def _matmul(x, v):
    bn = 2000
    return pl.pallas_call(
        _mm_body,
        grid=(_N // bn,),
        in_specs=[
            pl.BlockSpec((_BS, bn, _INC), lambda i: (0, i, 0)),
            pl.BlockSpec((_SEQ, _INC, _OUTC), lambda i: (0, 0, 0)),
        ],
        out_specs=pl.BlockSpec((_SEQ, bn, _LANES), lambda i: (0, i, 0)),
        out_shape=jax.ShapeDtypeStruct((_SEQ, _N, _LANES), jnp.float32),
    )(x, v)


def _gather_elu(ys3, idxf, bias4):
    mesh = plsc.VectorSubcoreMesh(core_axis_name="c", subcore_axis_name="s")

    @functools.partial(
        pl.kernel,
        out_type=jax.ShapeDtypeStruct((_NP, _LANES), jnp.float32),
        mesh=mesh,
        compiler_params=pltpu.CompilerParams(use_tc_tiling_on_sc=False),
        scratch_types=[
            pltpu.VMEM((2, _SEQ, _CH, _LANES), jnp.float32),
            pltpu.VMEM((2, _CH, _LANES), jnp.float32),
            pltpu.VMEM((_CPW0 * _SEQ * _CH,), jnp.int32),
            pltpu.VMEM((_LANES,), jnp.float32),
            pltpu.SemaphoreType.DMA,
            pltpu.SemaphoreType.DMA,
            pltpu.SemaphoreType.DMA,
            pltpu.SemaphoreType.DMA,
        ],
    )
    def k(ys_hbm, idx_hbm, b_hbm, h_hbm, gbuf, hbuf, idxv, biasv,
          sg0, sg1, sh0, sh1):
        cid = lax.axis_index("c")
        sid = lax.axis_index("s")
        gsems = [sg0, sg1]
        hsems = [sh0, sh1]
        pltpu.sync_copy(b_hbm, biasv)
        bvs = [biasv[pl.ds(16 * i, 16)] for i in range(_LANES // 16)]

        def run(ch0, nch):
            pltpu.sync_copy(
                idx_hbm.at[pl.ds(ch0 * _SEQ * _CH, nch * _SEQ * _CH)],
                idxv.at[pl.ds(0, nch * _SEQ * _CH)],
            )

            def fire(j):
                p = j % 2
                descs = []
                for s in range(_SEQ):
                    descs.append(
                        pltpu.async_copy(
                            ys_hbm.at[
                                idxv.at[pl.ds((j * _SEQ + s) * _CH, _CH)]
                            ],
                            gbuf.at[p, s],
                            gsems[p],
                        )
                    )
                return descs

            gdescs = {0: fire(0)}
            hdescs = {}
            for j in range(nch):
                p = j % 2
                ch = ch0 + j
                for d in gdescs.pop(j):
                    d.wait()
                if j + 1 < nch:
                    gdescs[j + 1] = fire(j + 1)
                if j - 2 in hdescs:
                    hdescs.pop(j - 2).wait()

                @pl.loop(0, _CH)
                def _(n):
                    for i in range(_LANES // 16):
                        acc = gbuf[p, 0, n, pl.ds(16 * i, 16)]
                        for s in range(1, _SEQ):
                            acc = acc + gbuf[p, s, n, pl.ds(16 * i, 16)]
                        acc = acc + bvs[i]
                        acc = jnp.where(acc > 0.0, acc, jnp.exp(acc) - 1.0)
                        hbuf[p, n, pl.ds(16 * i, 16)] = acc

                hdescs[j] = pltpu.async_copy(
                    hbuf.at[p], h_hbm.at[pl.ds(ch * _CH, _CH)], hsems[p]
                )
            for j, d in hdescs.items():
                d.wait()

        @pl.when(cid == 0)
        def _():
            run(sid * (2 * _CPW), _CPW0)

        @pl.when(cid == 1)
        def _():
            run(sid * (2 * _CPW) + _CPW0, _CPW1)

    return k(ys3, idxf, bias4)


def _bcast16(vec16, t):
    return lax.gather(
        vec16,
        jnp.full((16, 1), t, jnp.int32),
        lax.GatherDimensionNumbers(
            offset_dims=(),
            collapsed_slice_dims=(0,),
            start_index_map=(0,),
        ),
        (1,),
        mode=lax.GatherScatterMode.PROMISE_IN_BOUNDS,
    )


_RPT = _NDP // _NW
_RPT0 = 160
_RPT1 = 2 * _RPT - _RPT0


def _pool(h, colf, rowf, valf, tb):
    mesh = plsc.VectorSubcoreMesh(core_axis_name="c", subcore_axis_name="s")

    @functools.partial(
        pl.kernel,
        out_type=jax.ShapeDtypeStruct((_BS, _NDOWN, _OUTC), jnp.float32),
        mesh=mesh,
        compiler_params=pltpu.CompilerParams(
            use_tc_tiling_on_sc=False, needs_layout_passes=False
        ),
        scratch_types=[
            pltpu.VMEM((_RPT0, _LANES), jnp.float32),
            pltpu.VMEM((2, _KCH, _LANES), jnp.float32),
            pltpu.VMEM((_NKP,), jnp.int32),
            pltpu.VMEM((_NKP,), jnp.int32),
            pltpu.VMEM((_NKP,), jnp.float32),
            pltpu.VMEM((16,), jnp.int32),
            pltpu.SemaphoreType.DMA,
            pltpu.SemaphoreType.DMA,
            pltpu.SemaphoreType.DMA,
        ],
    )
    def k(h_hbm, col_hbm, row_hbm, val_hbm, tb_hbm, out_hbm,
          local, gbuf, colv, rowv, valv, tbv, semi, sg0, sg1):
        cid = lax.axis_index("c")
        sid = lax.axis_index("s")
        tid = sid * _NC + cid
        iota = lax.iota(jnp.int32, 16)

        di = [
            pltpu.async_copy(col_hbm, colv, semi),
            pltpu.async_copy(row_hbm, rowv, semi),
            pltpu.async_copy(val_hbm, valv, semi),
        ]
        pltpu.sync_copy(tb_hbm.at[pl.ds(tid * 16, 16)], tbv)

        zero = jnp.zeros((16,), jnp.float32)

        @pl.loop(0, _RPT0, unroll=4)
        def _(n):
            for i in range(_LANES // 16):
                local[n, pl.ds(16 * i, 16)] = zero

        for d in di:
            d.wait()
        tbvec = tbv[...]
        klo = jnp.max(jnp.where(iota == 0, tbvec, jnp.int32(-1)))
        khi = jnp.max(jnp.where(iota == 1, tbvec, jnp.int32(-1)))
        c_lo = lax.shift_right_logical(klo, 7)
        c_hi = lax.shift_right_logical(khi + 127, 7)
        rbase = sid * (2 * _RPT) + cid * _RPT0
        rmax = jnp.int32(_RPT0 - 1) - cid * jnp.int32(_RPT0 - _RPT1)
        @pl.when(c_lo < c_hi)
        def _():
            pltpu.async_copy(
                h_hbm.at[colv.at[pl.ds(c_lo * _KCH, _KCH)]],
                gbuf.at[0],
                sg0,
            )

        @pl.loop(c_lo, c_hi)
        def _(c):
            j = c - c_lo
            p = jnp.bitwise_and(j, 1)
            nxt = c + 1 < c_hi

            @pl.when(jnp.logical_and(nxt, p == 0))
            def _():
                pltpu.async_copy(
                    h_hbm.at[colv.at[pl.ds((c + 1) * _KCH, _KCH)]],
                    gbuf.at[1],
                    sg1,
                )

            @pl.when(jnp.logical_and(nxt, p == 1))
            def _():
                pltpu.async_copy(
                    h_hbm.at[colv.at[pl.ds((c + 1) * _KCH, _KCH)]],
                    gbuf.at[0],
                    sg0,
                )

            @pl.when(p == 0)
            def _():
                pltpu.make_async_copy(
                    h_hbm.at[colv.at[pl.ds(0, _KCH)]], gbuf.at[0], sg0
                ).wait()

            @pl.when(p == 1)
            def _():
                pltpu.make_async_copy(
                    h_hbm.at[colv.at[pl.ds(0, _KCH)]], gbuf.at[1], sg1
                ).wait()

            for g in range(_KCH // 16):
                kvec = c * _KCH + 16 * g + iota
                ok = jnp.logical_and(kvec >= klo, kvec < khi)
                val16 = jnp.where(
                    ok, valv[pl.ds(c * _KCH + 16 * g, 16)], 0.0
                )
                rows16 = rowv[pl.ds(c * _KCH + 16 * g, 16)] - rbase
                rows16 = jnp.minimum(jnp.maximum(rows16, 0), rmax)

                @pl.loop(0, 16)
                def _(t):
                    vb = _bcast16(val16, t)
                    rb = _bcast16(rows16, t)
                    for i in range(_LANES // 16):
                        data = gbuf[p, 16 * g + t, pl.ds(16 * i, 16)] * vb
                        plsc.addupdate_scatter(
                            local, [rb, 16 * i + iota], data
                        )

        @pl.when(cid == 0)
        def _():
            for b in range(_BS):
                pltpu.sync_copy(
                    local.at[pl.ds(0, _RPT0), pl.ds(b * _OUTC, _OUTC)],
                    out_hbm.at[b, pl.ds(rbase, _RPT0)],
                )

        @pl.when(jnp.logical_and(cid == 1, sid < _NS - 1))
        def _():
            for b in range(_BS):
                pltpu.sync_copy(
                    local.at[pl.ds(0, _RPT1), pl.ds(b * _OUTC, _OUTC)],
                    out_hbm.at[b, pl.ds(rbase, _RPT1)],
                )

        @pl.when(jnp.logical_and(cid == 1, sid == _NS - 1))
        def _():
            tail = _NDOWN - ((_NS - 1) * 2 * _RPT + _RPT0)
            for b in range(_BS):
                pltpu.sync_copy(
                    local.at[pl.ds(0, tail), pl.ds(b * _OUTC, _OUTC)],
                    out_hbm.at[b, pl.ds(rbase, tail)],
                )

    return k(h, colf, rowf, valf, tb)


def kernel(x, indices, down_row, down_col, down_val, W, b):
    v = W.reshape(_OUTC, _SEQ, _INC).transpose(1, 2, 0)
    ys3 = _matmul(x, v).reshape(_SEQ * _N, _LANES)

    idxt = indices.astype(jnp.int32).T
    idxt = jnp.pad(idxt, ((0, 0), (0, _NP - _N)))
    idxa = idxt + (jnp.arange(_SEQ, dtype=jnp.int32) * _N)[:, None]
    idxf = idxa.reshape(_SEQ, _NCHUNK, _CH).transpose(1, 0, 2).reshape(-1)

    bias4 = jnp.tile(b, _BS)
    h = _gather_elu(ys3, idxf, bias4)

    colf = jnp.pad(down_col.astype(jnp.int32), (0, _NKP - _NNZ))
    rowf = jnp.pad(
        down_row.astype(jnp.int32), (0, _NKP - _NNZ),
        constant_values=_NDOWN,
    )
    valf = jnp.pad(down_val, (0, _NKP - _NNZ))
    tids = jnp.arange(_NW, dtype=jnp.int32)
    starts = (tids // 2) * (2 * _RPT) + (tids % 2) * _RPT0
    sizes = jnp.where(tids % 2 == 0, _RPT0, _RPT1)
    lo = jnp.searchsorted(rowf, starts, side="left").astype(jnp.int32)
    hi = jnp.searchsorted(rowf, starts + sizes, side="left").astype(jnp.int32)
    tb = jnp.pad(jnp.stack([lo, hi], axis=1), ((0, 0), (0, 14))).reshape(-1)

    return _pool(h, colf, rowf, valf, tb)


# speedup vs baseline: 1.0248x; 1.0248x over previous
import functools

import jax
import jax.numpy as jnp
from jax import lax
from jax.experimental import pallas as pl
from jax.experimental.pallas import tpu as pltpu
from jax.experimental.pallas import tpu_sc as plsc

_BS = 4
_N = 10000
_SEQ = 9
_INC = 128
_OUTC = 32
_LANES = _BS * _OUTC
_NDOWN = 5000
_NNZ = 20000

_NC = 2
_NS = 16
_NW = _NC * _NS

_CH = 40
_CPW = 8
_NP = _NW * _CPW * _CH
_NCHUNK = _NP // _CH
_CPW0 = 9
_CPW1 = 2 * _CPW - _CPW0

_KCH = 128
_KPT = 5
_NKCH = _NC * _NS * _KPT
_NKP = _NKCH * _KCH
_NDP = 5120


def _mm_body(x_ref, v_ref, o_ref):
    for s in range(_SEQ):
        for bb in range(_BS):
            o_ref[s, :, bb * _OUTC:(bb + 1) * _OUTC] = jnp.dot(
                x_ref[bb], v_ref[s], preferred_element_type=jnp.float32
            )


def _matmul(x, v):
    bn = 2000
    return pl.pallas_call(
        _mm_body,
        grid=(_N // bn,),
        in_specs=[
            pl.BlockSpec((_BS, bn, _INC), lambda i: (0, i, 0)),
            pl.BlockSpec((_SEQ, _INC, _OUTC), lambda i: (0, 0, 0)),
        ],
        out_specs=pl.BlockSpec((_SEQ, bn, _LANES), lambda i: (0, i, 0)),
        out_shape=jax.ShapeDtypeStruct((_SEQ, _N, _LANES), jnp.float32),
    )(x, v)


def _gather_elu(ys3, idxf, bias4):
    mesh = plsc.VectorSubcoreMesh(core_axis_name="c", subcore_axis_name="s")

    @functools.partial(
        pl.kernel,
        out_type=jax.ShapeDtypeStruct((_NP, _LANES), jnp.float32),
        mesh=mesh,
        compiler_params=pltpu.CompilerParams(use_tc_tiling_on_sc=False),
        scratch_types=[
            pltpu.VMEM((2, _SEQ, _CH, _LANES), jnp.float32),
            pltpu.VMEM((2, _CH, _LANES), jnp.float32),
            pltpu.VMEM((_CPW0 * _SEQ * _CH,), jnp.int32),
            pltpu.VMEM((_LANES,), jnp.float32),
            pltpu.SemaphoreType.DMA,
            pltpu.SemaphoreType.DMA,
            pltpu.SemaphoreType.DMA,
            pltpu.SemaphoreType.DMA,
        ],
    )
    def k(ys_hbm, idx_hbm, b_hbm, h_hbm, gbuf, hbuf, idxv, biasv,
          sg0, sg1, sh0, sh1):
        cid = lax.axis_index("c")
        sid = lax.axis_index("s")
        gsems = [sg0, sg1]
        hsems = [sh0, sh1]
        pltpu.sync_copy(b_hbm, biasv)
        bvs = [biasv[pl.ds(16 * i, 16)] for i in range(_LANES // 16)]

        def run(ch0, nch):
            pltpu.sync_copy(
                idx_hbm.at[pl.ds(ch0 * _SEQ * _CH, nch * _SEQ * _CH)],
                idxv.at[pl.ds(0, nch * _SEQ * _CH)],
            )

            def fire(j):
                p = j % 2
                descs = []
                for s in range(_SEQ):
                    descs.append(
                        pltpu.async_copy(
                            ys_hbm.at[
                                idxv.at[pl.ds((j * _SEQ + s) * _CH, _CH)]
                            ],
                            gbuf.at[p, s],
                            gsems[p],
                        )
                    )
                return descs

            gdescs = {0: fire(0)}
            hdescs = {}
            for j in range(nch):
                p = j % 2
                ch = ch0 + j
                for d in gdescs.pop(j):
                    d.wait()
                if j + 1 < nch:
                    gdescs[j + 1] = fire(j + 1)
                if j - 2 in hdescs:
                    hdescs.pop(j - 2).wait()

                @pl.loop(0, _CH)
                def _(n):
                    for i in range(_LANES // 16):
                        acc = gbuf[p, 0, n, pl.ds(16 * i, 16)]
                        for s in range(1, _SEQ):
                            acc = acc + gbuf[p, s, n, pl.ds(16 * i, 16)]
                        acc = acc + bvs[i]
                        acc = jnp.where(acc > 0.0, acc, jnp.exp(acc) - 1.0)
                        hbuf[p, n, pl.ds(16 * i, 16)] = acc

                hdescs[j] = pltpu.async_copy(
                    hbuf.at[p], h_hbm.at[pl.ds(ch * _CH, _CH)], hsems[p]
                )
            for j, d in hdescs.items():
                d.wait()

        @pl.when(cid == 0)
        def _():
            run(sid * (2 * _CPW), _CPW0)

        @pl.when(cid == 1)
        def _():
            run(sid * (2 * _CPW) + _CPW0, _CPW1)

    return k(ys3, idxf, bias4)


def _bcast16(vec16, t):
    return lax.gather(
        vec16,
        jnp.full((16, 1), t, jnp.int32),
        lax.GatherDimensionNumbers(
            offset_dims=(),
            collapsed_slice_dims=(0,),
            start_index_map=(0,),
        ),
        (1,),
        mode=lax.GatherScatterMode.PROMISE_IN_BOUNDS,
    )


_RPT = _NDP // _NW
_RPT0 = 176
_RPT1 = 2 * _RPT - _RPT0


def _pool(h, colf, rowf, valf, tb):
    mesh = plsc.VectorSubcoreMesh(core_axis_name="c", subcore_axis_name="s")

    @functools.partial(
        pl.kernel,
        out_type=jax.ShapeDtypeStruct((_BS, _NDOWN, _OUTC), jnp.float32),
        mesh=mesh,
        compiler_params=pltpu.CompilerParams(
            use_tc_tiling_on_sc=False, needs_layout_passes=False
        ),
        scratch_types=[
            pltpu.VMEM((_RPT0, _LANES), jnp.float32),
            pltpu.VMEM((2, _KCH, _LANES), jnp.float32),
            pltpu.VMEM((_NKP,), jnp.int32),
            pltpu.VMEM((_NKP,), jnp.int32),
            pltpu.VMEM((_NKP,), jnp.float32),
            pltpu.VMEM((16,), jnp.int32),
            pltpu.SemaphoreType.DMA,
            pltpu.SemaphoreType.DMA,
            pltpu.SemaphoreType.DMA,
        ],
    )
    def k(h_hbm, col_hbm, row_hbm, val_hbm, tb_hbm, out_hbm,
          local, gbuf, colv, rowv, valv, tbv, semi, sg0, sg1):
        cid = lax.axis_index("c")
        sid = lax.axis_index("s")
        tid = sid * _NC + cid
        iota = lax.iota(jnp.int32, 16)

        di = [
            pltpu.async_copy(col_hbm, colv, semi),
            pltpu.async_copy(row_hbm, rowv, semi),
            pltpu.async_copy(val_hbm, valv, semi),
        ]
        pltpu.sync_copy(tb_hbm.at[pl.ds(tid * 16, 16)], tbv)

        zero = jnp.zeros((16,), jnp.float32)

        @pl.loop(0, _RPT0, unroll=4)
        def _(n):
            for i in range(_LANES // 16):
                local[n, pl.ds(16 * i, 16)] = zero

        for d in di:
            d.wait()
        tbvec = tbv[...]
        klo = jnp.max(jnp.where(iota == 0, tbvec, jnp.int32(-1)))
        khi = jnp.max(jnp.where(iota == 1, tbvec, jnp.int32(-1)))
        c_lo = lax.shift_right_logical(klo, 7)
        c_hi = lax.shift_right_logical(khi + 127, 7)
        rbase = sid * (2 * _RPT) + cid * _RPT0
        rmax = jnp.int32(_RPT0 - 1) - cid * jnp.int32(_RPT0 - _RPT1)
        @pl.when(c_lo < c_hi)
        def _():
            pltpu.async_copy(
                h_hbm.at[colv.at[pl.ds(c_lo * _KCH, _KCH)]],
                gbuf.at[0],
                sg0,
            )

        @pl.loop(c_lo, c_hi)
        def _(c):
            j = c - c_lo
            p = jnp.bitwise_and(j, 1)
            nxt = c + 1 < c_hi

            @pl.when(jnp.logical_and(nxt, p == 0))
            def _():
                pltpu.async_copy(
                    h_hbm.at[colv.at[pl.ds((c + 1) * _KCH, _KCH)]],
                    gbuf.at[1],
                    sg1,
                )

            @pl.when(jnp.logical_and(nxt, p == 1))
            def _():
                pltpu.async_copy(
                    h_hbm.at[colv.at[pl.ds((c + 1) * _KCH, _KCH)]],
                    gbuf.at[0],
                    sg0,
                )

            @pl.when(p == 0)
            def _():
                pltpu.make_async_copy(
                    h_hbm.at[colv.at[pl.ds(0, _KCH)]], gbuf.at[0], sg0
                ).wait()

            @pl.when(p == 1)
            def _():
                pltpu.make_async_copy(
                    h_hbm.at[colv.at[pl.ds(0, _KCH)]], gbuf.at[1], sg1
                ).wait()

            for g in range(_KCH // 16):
                kvec = c * _KCH + 16 * g + iota
                ok = jnp.logical_and(kvec >= klo, kvec < khi)
                val16 = jnp.where(
                    ok, valv[pl.ds(c * _KCH + 16 * g, 16)], 0.0
                )
                rows16 = rowv[pl.ds(c * _KCH + 16 * g, 16)] - rbase
                rows16 = jnp.minimum(jnp.maximum(rows16, 0), rmax)

                @pl.loop(0, 16)
                def _(t):
                    vb = _bcast16(val16, t)
                    rb = _bcast16(rows16, t)
                    for i in range(_LANES // 16):
                        data = gbuf[p, 16 * g + t, pl.ds(16 * i, 16)] * vb
                        plsc.addupdate_scatter(
                            local, [rb, 16 * i + iota], data
                        )

        @pl.when(cid == 0)
        def _():
            for b in range(_BS):
                pltpu.sync_copy(
                    local.at[pl.ds(0, _RPT0), pl.ds(b * _OUTC, _OUTC)],
                    out_hbm.at[b, pl.ds(rbase, _RPT0)],
                )

        @pl.when(jnp.logical_and(cid == 1, sid < _NS - 1))
        def _():
            for b in range(_BS):
                pltpu.sync_copy(
                    local.at[pl.ds(0, _RPT1), pl.ds(b * _OUTC, _OUTC)],
                    out_hbm.at[b, pl.ds(rbase, _RPT1)],
                )

        @pl.when(jnp.logical_and(cid == 1, sid == _NS - 1))
        def _():
            tail = _NDOWN - ((_NS - 1) * 2 * _RPT + _RPT0)
            for b in range(_BS):
                pltpu.sync_copy(
                    local.at[pl.ds(0, tail), pl.ds(b * _OUTC, _OUTC)],
                    out_hbm.at[b, pl.ds(rbase, tail)],
                )

    return k(h, colf, rowf, valf, tb)


def kernel(x, indices, down_row, down_col, down_val, W, b):
    v = W.reshape(_OUTC, _SEQ, _INC).transpose(1, 2, 0)
    ys3 = _matmul(x, v).reshape(_SEQ * _N, _LANES)

    idxt = indices.astype(jnp.int32).T
    idxt = jnp.pad(idxt, ((0, 0), (0, _NP - _N)))
    idxa = idxt + (jnp.arange(_SEQ, dtype=jnp.int32) * _N)[:, None]
    idxf = idxa.reshape(_SEQ, _NCHUNK, _CH).transpose(1, 0, 2).reshape(-1)

    bias4 = jnp.tile(b, _BS)
    h = _gather_elu(ys3, idxf, bias4)

    colf = jnp.pad(down_col.astype(jnp.int32), (0, _NKP - _NNZ))
    rowf = jnp.pad(
        down_row.astype(jnp.int32), (0, _NKP - _NNZ),
        constant_values=_NDOWN,
    )
    valf = jnp.pad(down_val, (0, _NKP - _NNZ))
    tids = jnp.arange(_NW, dtype=jnp.int32)
    starts = (tids // 2) * (2 * _RPT) + (tids % 2) * _RPT0
    sizes = jnp.where(tids % 2 == 0, _RPT0, _RPT1)
    lo = jnp.searchsorted(rowf, starts, side="left").astype(jnp.int32)
    hi = jnp.searchsorted(rowf, starts + sizes, side="left").astype(jnp.int32)
    tb = jnp.pad(jnp.stack([lo, hi], axis=1), ((0, 0), (0, 14))).reshape(-1)

    return _pool(h, colf, rowf, valf, tb)


# speedup vs baseline: 1.0340x; 1.0090x over previous
import functools

import jax
import jax.numpy as jnp
from jax import lax
from jax.experimental import pallas as pl
from jax.experimental.pallas import tpu as pltpu
from jax.experimental.pallas import tpu_sc as plsc

_BS = 4
_N = 10000
_SEQ = 9
_INC = 128
_OUTC = 32
_LANES = _BS * _OUTC
_NDOWN = 5000
_NNZ = 20000

_NC = 2
_NS = 16
_NW = _NC * _NS

_CH = 40
_CPW = 8
_NP = _NW * _CPW * _CH
_NCHUNK = _NP // _CH
_CPW0 = 10
_CPW1 = 2 * _CPW - _CPW0

_KCH = 128
_KPT = 5
_NKCH = _NC * _NS * _KPT
_NKP = _NKCH * _KCH
_NDP = 5120


def _mm_body(x_ref, v_ref, o_ref):
    for s in range(_SEQ):
        for bb in range(_BS):
            o_ref[s, :, bb * _OUTC:(bb + 1) * _OUTC] = jnp.dot(
                x_ref[bb], v_ref[s], preferred_element_type=jnp.float32
            )


def _matmul(x, v):
    bn = 2000
    return pl.pallas_call(
        _mm_body,
        grid=(_N // bn,),
        in_specs=[
            pl.BlockSpec((_BS, bn, _INC), lambda i: (0, i, 0)),
            pl.BlockSpec((_SEQ, _INC, _OUTC), lambda i: (0, 0, 0)),
        ],
        out_specs=pl.BlockSpec((_SEQ, bn, _LANES), lambda i: (0, i, 0)),
        out_shape=jax.ShapeDtypeStruct((_SEQ, _N, _LANES), jnp.float32),
    )(x, v)


def _gather_elu(ys3, idxf, bias4):
    mesh = plsc.VectorSubcoreMesh(core_axis_name="c", subcore_axis_name="s")

    @functools.partial(
        pl.kernel,
        out_type=jax.ShapeDtypeStruct((_NP, _LANES), jnp.float32),
        mesh=mesh,
        compiler_params=pltpu.CompilerParams(use_tc_tiling_on_sc=False),
        scratch_types=[
            pltpu.VMEM((2, _SEQ, _CH, _LANES), jnp.float32),
            pltpu.VMEM((2, _CH, _LANES), jnp.float32),
            pltpu.VMEM((_CPW0 * _SEQ * _CH,), jnp.int32),
            pltpu.VMEM((_LANES,), jnp.float32),
            pltpu.SemaphoreType.DMA,
            pltpu.SemaphoreType.DMA,
            pltpu.SemaphoreType.DMA,
            pltpu.SemaphoreType.DMA,
        ],
    )
    def k(ys_hbm, idx_hbm, b_hbm, h_hbm, gbuf, hbuf, idxv, biasv,
          sg0, sg1, sh0, sh1):
        cid = lax.axis_index("c")
        sid = lax.axis_index("s")
        gsems = [sg0, sg1]
        hsems = [sh0, sh1]
        pltpu.sync_copy(b_hbm, biasv)
        bvs = [biasv[pl.ds(16 * i, 16)] for i in range(_LANES // 16)]

        def run(ch0, nch):
            pltpu.sync_copy(
                idx_hbm.at[pl.ds(ch0 * _SEQ * _CH, nch * _SEQ * _CH)],
                idxv.at[pl.ds(0, nch * _SEQ * _CH)],
            )

            def fire(j):
                p = j % 2
                descs = []
                for s in range(_SEQ):
                    descs.append(
                        pltpu.async_copy(
                            ys_hbm.at[
                                idxv.at[pl.ds((j * _SEQ + s) * _CH, _CH)]
                            ],
                            gbuf.at[p, s],
                            gsems[p],
                        )
                    )
                return descs

            gdescs = {0: fire(0)}
            hdescs = {}
            for j in range(nch):
                p = j % 2
                ch = ch0 + j
                for d in gdescs.pop(j):
                    d.wait()
                if j + 1 < nch:
                    gdescs[j + 1] = fire(j + 1)
                if j - 2 in hdescs:
                    hdescs.pop(j - 2).wait()

                @pl.loop(0, _CH)
                def _(n):
                    for i in range(_LANES // 16):
                        acc = gbuf[p, 0, n, pl.ds(16 * i, 16)]
                        for s in range(1, _SEQ):
                            acc = acc + gbuf[p, s, n, pl.ds(16 * i, 16)]
                        acc = acc + bvs[i]
                        acc = jnp.where(acc > 0.0, acc, jnp.exp(acc) - 1.0)
                        hbuf[p, n, pl.ds(16 * i, 16)] = acc

                hdescs[j] = pltpu.async_copy(
                    hbuf.at[p], h_hbm.at[pl.ds(ch * _CH, _CH)], hsems[p]
                )
            for j, d in hdescs.items():
                d.wait()

        @pl.when(cid == 0)
        def _():
            run(sid * (2 * _CPW), _CPW0)

        @pl.when(cid == 1)
        def _():
            run(sid * (2 * _CPW) + _CPW0, _CPW1)

    return k(ys3, idxf, bias4)


def _bcast16(vec16, t):
    return lax.gather(
        vec16,
        jnp.full((16, 1), t, jnp.int32),
        lax.GatherDimensionNumbers(
            offset_dims=(),
            collapsed_slice_dims=(0,),
            start_index_map=(0,),
        ),
        (1,),
        mode=lax.GatherScatterMode.PROMISE_IN_BOUNDS,
    )


_RPT = _NDP // _NW
_RPT0 = 192
_RPT1 = 2 * _RPT - _RPT0


def _pool(h, colf, rowf, valf, tb):
    mesh = plsc.VectorSubcoreMesh(core_axis_name="c", subcore_axis_name="s")

    @functools.partial(
        pl.kernel,
        out_type=jax.ShapeDtypeStruct((_BS, _NDOWN, _OUTC), jnp.float32),
        mesh=mesh,
        compiler_params=pltpu.CompilerParams(
            use_tc_tiling_on_sc=False, needs_layout_passes=False
        ),
        scratch_types=[
            pltpu.VMEM((_RPT0, _LANES), jnp.float32),
            pltpu.VMEM((2, _KCH, _LANES), jnp.float32),
            pltpu.VMEM((_NKP,), jnp.int32),
            pltpu.VMEM((_NKP,), jnp.int32),
            pltpu.VMEM((_NKP,), jnp.float32),
            pltpu.VMEM((16,), jnp.int32),
            pltpu.SemaphoreType.DMA,
            pltpu.SemaphoreType.DMA,
            pltpu.SemaphoreType.DMA,
        ],
    )
    def k(h_hbm, col_hbm, row_hbm, val_hbm, tb_hbm, out_hbm,
          local, gbuf, colv, rowv, valv, tbv, semi, sg0, sg1):
        cid = lax.axis_index("c")
        sid = lax.axis_index("s")
        tid = sid * _NC + cid
        iota = lax.iota(jnp.int32, 16)

        di = [
            pltpu.async_copy(col_hbm, colv, semi),
            pltpu.async_copy(row_hbm, rowv, semi),
            pltpu.async_copy(val_hbm, valv, semi),
        ]
        pltpu.sync_copy(tb_hbm.at[pl.ds(tid * 16, 16)], tbv)

        zero = jnp.zeros((16,), jnp.float32)

        @pl.loop(0, _RPT0, unroll=4)
        def _(n):
            for i in range(_LANES // 16):
                local[n, pl.ds(16 * i, 16)] = zero

        for d in di:
            d.wait()
        tbvec = tbv[...]
        klo = jnp.max(jnp.where(iota == 0, tbvec, jnp.int32(-1)))
        khi = jnp.max(jnp.where(iota == 1, tbvec, jnp.int32(-1)))
        c_lo = lax.shift_right_logical(klo, 7)
        c_hi = lax.shift_right_logical(khi + 127, 7)
        rbase = sid * (2 * _RPT) + cid * _RPT0
        rmax = jnp.int32(_RPT0 - 1) - cid * jnp.int32(_RPT0 - _RPT1)
        @pl.when(c_lo < c_hi)
        def _():
            pltpu.async_copy(
                h_hbm.at[colv.at[pl.ds(c_lo * _KCH, _KCH)]],
                gbuf.at[0],
                sg0,
            )

        @pl.loop(c_lo, c_hi)
        def _(c):
            j = c - c_lo
            p = jnp.bitwise_and(j, 1)
            nxt = c + 1 < c_hi

            @pl.when(jnp.logical_and(nxt, p == 0))
            def _():
                pltpu.async_copy(
                    h_hbm.at[colv.at[pl.ds((c + 1) * _KCH, _KCH)]],
                    gbuf.at[1],
                    sg1,
                )

            @pl.when(jnp.logical_and(nxt, p == 1))
            def _():
                pltpu.async_copy(
                    h_hbm.at[colv.at[pl.ds((c + 1) * _KCH, _KCH)]],
                    gbuf.at[0],
                    sg0,
                )

            @pl.when(p == 0)
            def _():
                pltpu.make_async_copy(
                    h_hbm.at[colv.at[pl.ds(0, _KCH)]], gbuf.at[0], sg0
                ).wait()

            @pl.when(p == 1)
            def _():
                pltpu.make_async_copy(
                    h_hbm.at[colv.at[pl.ds(0, _KCH)]], gbuf.at[1], sg1
                ).wait()

            for g in range(_KCH // 16):
                kvec = c * _KCH + 16 * g + iota
                ok = jnp.logical_and(kvec >= klo, kvec < khi)
                val16 = jnp.where(
                    ok, valv[pl.ds(c * _KCH + 16 * g, 16)], 0.0
                )
                rows16 = rowv[pl.ds(c * _KCH + 16 * g, 16)] - rbase
                rows16 = jnp.minimum(jnp.maximum(rows16, 0), rmax)

                @pl.loop(0, 16)
                def _(t):
                    vb = _bcast16(val16, t)
                    rb = _bcast16(rows16, t)
                    for i in range(_LANES // 16):
                        data = gbuf[p, 16 * g + t, pl.ds(16 * i, 16)] * vb
                        plsc.addupdate_scatter(
                            local, [rb, 16 * i + iota], data
                        )

        @pl.when(cid == 0)
        def _():
            for b in range(_BS):
                pltpu.sync_copy(
                    local.at[pl.ds(0, _RPT0), pl.ds(b * _OUTC, _OUTC)],
                    out_hbm.at[b, pl.ds(rbase, _RPT0)],
                )

        @pl.when(jnp.logical_and(cid == 1, sid < _NS - 1))
        def _():
            for b in range(_BS):
                pltpu.sync_copy(
                    local.at[pl.ds(0, _RPT1), pl.ds(b * _OUTC, _OUTC)],
                    out_hbm.at[b, pl.ds(rbase, _RPT1)],
                )

        @pl.when(jnp.logical_and(cid == 1, sid == _NS - 1))
        def _():
            tail = _NDOWN - ((_NS - 1) * 2 * _RPT + _RPT0)
            for b in range(_BS):
                pltpu.sync_copy(
                    local.at[pl.ds(0, tail), pl.ds(b * _OUTC, _OUTC)],
                    out_hbm.at[b, pl.ds(rbase, tail)],
                )

    return k(h, colf, rowf, valf, tb)


def kernel(x, indices, down_row, down_col, down_val, W, b):
    v = W.reshape(_OUTC, _SEQ, _INC).transpose(1, 2, 0)
    ys3 = _matmul(x, v).reshape(_SEQ * _N, _LANES)

    idxt = indices.astype(jnp.int32).T
    idxt = jnp.pad(idxt, ((0, 0), (0, _NP - _N)))
    idxa = idxt + (jnp.arange(_SEQ, dtype=jnp.int32) * _N)[:, None]
    idxf = idxa.reshape(_SEQ, _NCHUNK, _CH).transpose(1, 0, 2).reshape(-1)

    bias4 = jnp.tile(b, _BS)
    h = _gather_elu(ys3, idxf, bias4)

    colf = jnp.pad(down_col.astype(jnp.int32), (0, _NKP - _NNZ))
    rowf = jnp.pad(
        down_row.astype(jnp.int32), (0, _NKP - _NNZ),
        constant_values=_NDOWN,
    )
    valf = jnp.pad(down_val, (0, _NKP - _NNZ))
    tids = jnp.arange(_NW, dtype=jnp.int32)
    starts = (tids // 2) * (2 * _RPT) + (tids % 2) * _RPT0
    sizes = jnp.where(tids % 2 == 0, _RPT0, _RPT1)
    lo = jnp.searchsorted(rowf, starts, side="left").astype(jnp.int32)
    hi = jnp.searchsorted(rowf, starts + sizes, side="left").astype(jnp.int32)
    tb = jnp.pad(jnp.stack([lo, hi], axis=1), ((0, 0), (0, 14))).reshape(-1)

    return _pool(h, colf, rowf, valf, tb)
